# baseline (speedup 1.0000x reference)
import functools

import jax
import jax.numpy as jnp
from jax import lax
from jax.experimental import pallas as pl
from jax.experimental.pallas import tpu as pltpu
from jax.experimental.pallas import tpu_sc as plsc

_TOPK = 8
_NUM_EXPERTS = 64
_BLOCK_ROWS = 1024
_NW = 32
_LANES = 16


def _score_kernel(x_ref, e_ref, b_ref, s_ref, st_ref):
    logits_t = jax.lax.dot_general(
        e_ref[...], x_ref[...],
        dimension_numbers=(((1,), (1,)), ((), ())),
        preferred_element_type=jnp.float32,
    )
    scores_t = jax.nn.sigmoid(logits_t) + b_ref[...]
    s_ref[...] = scores_t.T
    st_ref[0] = scores_t


def _sc_topk_kernel(st_hbm, wt_hbm, it_hbm, st_v, wv_v, iv_v, rpw):
    wid = lax.axis_index("s") * 2 + lax.axis_index("c")
    pltpu.sync_copy(st_hbm.at[wid], st_v)
    neg_inf = jnp.float32(-jnp.inf)

    def group(g, carry):
        base = g * _LANES
        vals = [st_v[e, pl.ds(base, _LANES)] for e in range(_NUM_EXPERTS)]
        total = None
        tops = []
        for _ in range(_TOPK):
            m = vals[0]
            for e in range(1, _NUM_EXPERTS):
                m = jnp.maximum(m, vals[e])
            idx = jnp.full((_LANES,), _NUM_EXPERTS, jnp.int32)
            for e in range(_NUM_EXPERTS - 1, -1, -1):
                idx = jnp.where(vals[e] == m, e, idx)
            for e in range(_NUM_EXPERTS):
                vals[e] = jnp.where(idx == e, neg_inf, vals[e])
            tops.append((m, idx))
            total = m if total is None else total + m
        for k, (m, idx) in enumerate(tops):
            wv_v[k, pl.ds(base, _LANES)] = m / total
            iv_v[k, pl.ds(base, _LANES)] = idx
        return carry

    lax.fori_loop(0, rpw // _LANES, group, 0)
    pltpu.sync_copy(wv_v, wt_hbm.at[wid])
    pltpu.sync_copy(iv_v, it_hbm.at[wid])


_CHUNKS = 4


@jax.jit
def kernel(x, expert_embeddings, bias):
    n_rows, n_cols = x.shape
    n_exp = expert_embeddings.shape[0]
    crows = n_rows // _CHUNKS
    rpw = crows // _NW
    grid = (crows // _BLOCK_ROWS,)
    bias2d = bias.reshape(n_exp, 1)
    mesh = plsc.VectorSubcoreMesh(core_axis_name="c", subcore_axis_name="s")

    score_call = pl.pallas_call(
        _score_kernel,
        grid=grid,
        in_specs=[
            pl.BlockSpec((_BLOCK_ROWS, n_cols), lambda i: (i, 0)),
            pl.BlockSpec((n_exp, n_cols), lambda i: (0, 0)),
            pl.BlockSpec((n_exp, 1), lambda i: (0, 0)),
        ],
        out_specs=[
            pl.BlockSpec((_BLOCK_ROWS, n_exp), lambda i: (i, 0)),
            pl.BlockSpec((1, n_exp, _BLOCK_ROWS), lambda i: (i, 0, 0)),
        ],
        out_shape=[
            jax.ShapeDtypeStruct((crows, n_exp), jnp.float32),
            jax.ShapeDtypeStruct((_NW, n_exp, rpw), jnp.float32),
        ],
    )

    topk_call = pl.kernel(
        functools.partial(_sc_topk_kernel, rpw=rpw),
        mesh=mesh,
        out_type=[
            jax.ShapeDtypeStruct((_NW, _TOPK, rpw), jnp.float32),
            jax.ShapeDtypeStruct((_NW, _TOPK, rpw), jnp.int32),
        ],
        scratch_types=[
            pltpu.VMEM((n_exp, rpw), jnp.float32),
            pltpu.VMEM((_TOPK, rpw), jnp.float32),
            pltpu.VMEM((_TOPK, rpw), jnp.int32),
        ],
    )

    scores_c, w_c, i_c = [], [], []
    for c in range(_CHUNKS):
        xc = jax.lax.slice_in_dim(x, c * crows, (c + 1) * crows, axis=0)
        sc, st = score_call(xc, expert_embeddings, bias2d)
        wt, it = topk_call(st)
        scores_c.append(sc)
        w_c.append(wt.transpose(0, 2, 1).reshape(crows, _TOPK))
        i_c.append(it.transpose(0, 2, 1).reshape(crows, _TOPK))

    scores = jnp.concatenate(scores_c, axis=0)
    weights = jnp.concatenate(w_c, axis=0)
    indices = jnp.concatenate(i_c, axis=0)
    return weights.astype(x.dtype), indices, scores


# speedup vs baseline: 2.6775x; 2.6775x over previous
import jax
import jax.numpy as jnp
from jax.experimental import pallas as pl

_TOPK = 8
_NUM_EXPERTS = 64
_BLOCK_ROWS = 1024


def _gate_kernel(x_ref, e_ref, b_ref, w_ref, i_ref, s_ref):
    logits_t = jax.lax.dot_general(
        e_ref[...], x_ref[...],
        dimension_numbers=(((1,), (1,)), ((), ())),
        preferred_element_type=jnp.float32,
    )
    scores_t = jax.nn.sigmoid(logits_t) + b_ref[...]
    s_ref[...] = scores_t.T

    iota = jax.lax.broadcasted_iota(jnp.int32, scores_t.shape, 0)
    vals = scores_t
    neg_inf = jnp.float32(-jnp.inf)
    top_v = []
    top_i = []
    for _ in range(_TOPK):
        m = jnp.max(vals, axis=0, keepdims=True)
        idx = jnp.min(jnp.where(vals == m, iota, _NUM_EXPERTS),
                      axis=0, keepdims=True)
        top_v.append(m)
        top_i.append(idx)
        vals = jnp.where(iota == idx, neg_inf, vals)
    v = jnp.concatenate(top_v, axis=0)
    i_ref[...] = jnp.concatenate(top_i, axis=0).T
    w_ref[...] = (v / jnp.sum(v, axis=0, keepdims=True)).T


@jax.jit
def kernel(x, expert_embeddings, bias):
    n_rows, _ = x.shape
    n_exp = expert_embeddings.shape[0]
    grid = (n_rows // _BLOCK_ROWS,)
    bias2d = bias.reshape(n_exp, 1)
    weights, indices, scores = pl.pallas_call(
        _gate_kernel,
        grid=grid,
        in_specs=[
            pl.BlockSpec((_BLOCK_ROWS, x.shape[1]), lambda i: (i, 0)),
            pl.BlockSpec((n_exp, x.shape[1]), lambda i: (0, 0)),
            pl.BlockSpec((n_exp, 1), lambda i: (0, 0)),
        ],
        out_specs=[
            pl.BlockSpec((_BLOCK_ROWS, _TOPK), lambda i: (i, 0)),
            pl.BlockSpec((_BLOCK_ROWS, _TOPK), lambda i: (i, 0)),
            pl.BlockSpec((_BLOCK_ROWS, n_exp), lambda i: (i, 0)),
        ],
        out_shape=[
            jax.ShapeDtypeStruct((n_rows, _TOPK), jnp.float32),
            jax.ShapeDtypeStruct((n_rows, _TOPK), jnp.int32),
            jax.ShapeDtypeStruct((n_rows, n_exp), jnp.float32),
        ],
    )(x, expert_embeddings, bias2d)
    return weights.astype(x.dtype), indices, scores


# speedup vs baseline: 2.6912x; 1.0051x over previous
import jax
import jax.numpy as jnp
from jax.experimental import pallas as pl

_TOPK = 8
_NUM_EXPERTS = 64
_BLOCK_ROWS = 1024


def _gate_kernel(x_ref, e_ref, b_ref, w_ref, i_ref, s_ref):
    logits_t = jax.lax.dot_general(
        e_ref[...], x_ref[...],
        dimension_numbers=(((1,), (1,)), ((), ())),
        preferred_element_type=jnp.float32,
    )
    scores_t = jax.nn.sigmoid(logits_t) + b_ref[...].reshape(_NUM_EXPERTS, 1)
    s_ref[...] = scores_t.T

    iota = jax.lax.broadcasted_iota(jnp.int32, scores_t.shape, 0)
    vals = scores_t
    neg_inf = jnp.float32(-jnp.inf)
    top_v = []
    top_i = []
    for _ in range(_TOPK):
        m = jnp.max(vals, axis=0, keepdims=True)
        idx = jnp.min(jnp.where(vals == m, iota, _NUM_EXPERTS),
                      axis=0, keepdims=True)
        top_v.append(m)
        top_i.append(idx)
        vals = jnp.where(iota == idx, neg_inf, vals)
    v = jnp.concatenate(top_v, axis=0)
    i_ref[...] = jnp.concatenate(top_i, axis=0).T
    w_ref[...] = (v / jnp.sum(v, axis=0, keepdims=True)).T


@jax.jit
def kernel(x, expert_embeddings, bias):
    n_rows, _ = x.shape
    n_exp = expert_embeddings.shape[0]
    grid = (n_rows // _BLOCK_ROWS,)
    weights, indices, scores = pl.pallas_call(
        _gate_kernel,
        grid=grid,
        in_specs=[
            pl.BlockSpec((_BLOCK_ROWS, x.shape[1]), lambda i: (i, 0)),
            pl.BlockSpec((n_exp, x.shape[1]), lambda i: (0, 0)),
            pl.BlockSpec((n_exp,), lambda i: (0,)),
        ],
        out_specs=[
            pl.BlockSpec((_BLOCK_ROWS, _TOPK), lambda i: (i, 0)),
            pl.BlockSpec((_BLOCK_ROWS, _TOPK), lambda i: (i, 0)),
            pl.BlockSpec((_BLOCK_ROWS, n_exp), lambda i: (i, 0)),
        ],
        out_shape=[
            jax.ShapeDtypeStruct((n_rows, _TOPK), jnp.float32),
            jax.ShapeDtypeStruct((n_rows, _TOPK), jnp.int32),
            jax.ShapeDtypeStruct((n_rows, n_exp), jnp.float32),
        ],
    )(x, expert_embeddings, bias)
    return weights.astype(x.dtype), indices, scores
